# baseline (speedup 1.0000x reference)
import functools

import jax
import jax.numpy as jnp
from jax import lax
from jax.experimental import pallas as pl
from jax.experimental.pallas import tpu as pltpu
from jax.experimental.pallas import tpu_sc as plsc

MAX_LEN = 100000
D_MODEL = 64
BATCH = 16384

_info = plsc.get_sparse_core_info()
_NC, _NS = _info.num_cores, _info.num_subcores
_NW = _NC * _NS
_B_PER_W = BATCH // _NW
_K = 16


def _gather_body(table_hbm, idx_hbm, out_hbm, idx_v, rows_v, sem):
    wid = lax.axis_index("s") * _NC + lax.axis_index("c")
    base = wid * _B_PER_W
    pltpu.sync_copy(idx_hbm.at[pl.ds(base, _B_PER_W)], idx_v)

    @pl.loop(0, _B_PER_W, step=_K)
    def _fire_drain(i):
        v = idx_v[pl.ds(i, _K)]
        descs = [
            pltpu.async_copy(
                table_hbm.at[pl.ds(v[b], 1), :],
                rows_v.at[pl.ds(i + b, 1), :],
                sem,
            )
            for b in range(_K)
        ]
        for d in descs:
            d.wait()

    pltpu.sync_copy(rows_v, out_hbm.at[pl.ds(base, _B_PER_W)])


@jax.jit
def _sc_gather(table, idx):
    mesh = plsc.VectorSubcoreMesh(core_axis_name="c", subcore_axis_name="s")
    return pl.kernel(
        _gather_body,
        mesh=mesh,
        out_type=jax.ShapeDtypeStruct((BATCH, D_MODEL), jnp.float32),
        scratch_types=[
            pltpu.VMEM((_B_PER_W,), jnp.int32),
            pltpu.VMEM((_B_PER_W, D_MODEL), jnp.float32),
            pltpu.SemaphoreType.DMA,
        ],
        compiler_params=pltpu.CompilerParams(use_tc_tiling_on_sc=True),
    )(table, idx)


def kernel(window_position, window_size, pos_embedding):
    del window_size
    return _sc_gather(pos_embedding, window_position.astype(jnp.int32))


# speedup vs baseline: 1.6440x; 1.6440x over previous
import functools

import jax
import jax.numpy as jnp
from jax import lax
from jax.experimental import pallas as pl
from jax.experimental.pallas import tpu as pltpu
from jax.experimental.pallas import tpu_sc as plsc

MAX_LEN = 100000
D_MODEL = 64
BATCH = 16384

_info = plsc.get_sparse_core_info()
_NC, _NS = _info.num_cores, _info.num_subcores
_NW = _NC * _NS
_COLS_PER_W = D_MODEL // _NW
_CHUNK = 8192


def _gather_body(table_t_hbm, idx_hbm, out_t_hbm, col_v, idx_v, out_v, sem):
    wid = lax.axis_index("s") * _NC + lax.axis_index("c")
    pltpu.sync_copy(idx_hbm, idx_v)
    for ci in range(_COLS_PER_W):
        c = wid * _COLS_PER_W + ci
        pltpu.sync_copy(table_t_hbm.at[c], col_v)
        for off in range(0, BATCH, _CHUNK):

            @pl.loop(0, _CHUNK, step=16, unroll=8)
            def _gather16(j):
                iv = idx_v[pl.ds(off + j, 16)]
                out_v[pl.ds(j, 16)] = plsc.load_gather(col_v, [iv])

            pltpu.sync_copy(out_v, out_t_hbm.at[c, pl.ds(off, _CHUNK)])


@jax.jit
def _sc_gather(table_t, idx):
    mesh = plsc.VectorSubcoreMesh(core_axis_name="c", subcore_axis_name="s")
    return pl.kernel(
        _gather_body,
        mesh=mesh,
        out_type=jax.ShapeDtypeStruct((D_MODEL, BATCH), jnp.float32),
        scratch_types=[
            pltpu.VMEM((MAX_LEN,), jnp.float32),
            pltpu.VMEM((BATCH,), jnp.int32),
            pltpu.VMEM((_CHUNK,), jnp.float32),
            pltpu.SemaphoreType.DMA,
        ],
        compiler_params=pltpu.CompilerParams(
            use_tc_tiling_on_sc=True, needs_layout_passes=False
        ),
    )(table_t, idx)


def kernel(window_position, window_size, pos_embedding):
    del window_size
    out_t = _sc_gather(pos_embedding.T, window_position.astype(jnp.int32))
    return out_t.T


# speedup vs baseline: 2.4297x; 1.4779x over previous
import functools

import jax
import jax.numpy as jnp
from jax import lax
from jax.experimental import pallas as pl
from jax.experimental.pallas import tpu as pltpu
from jax.experimental.pallas import tpu_sc as plsc

MAX_LEN = 100000
D_MODEL = 64
BATCH = 16384

_info = plsc.get_sparse_core_info()
_NC, _NS = _info.num_cores, _info.num_subcores
_NW = _NC * _NS
_COLS_PER_W = D_MODEL // _NW
_OCHUNK = 4096
_NCHUNK = BATCH // _OCHUNK


def _gather_body(
    table_t_hbm, idx_hbm, out_t_hbm, col_v, idx_v, out0_v, out1_v, sem_idx, sem_col, sem_o0, sem_o1
):
    wid = lax.axis_index("s") * _NC + lax.axis_index("c")
    c0 = wid * _COLS_PER_W
    idx_cp = pltpu.async_copy(idx_hbm, idx_v, sem_idx)
    col_cp = pltpu.async_copy(table_t_hbm.at[c0], col_v, sem_col)
    idx_cp.wait()
    col_cp.wait()

    outbufs = (out0_v, out1_v)
    osems = (sem_o0, sem_o1)
    pending = [None, None]
    for ci in range(_COLS_PER_W):
        c = c0 + ci
        for k in range(_NCHUNK):
            b = k % 2
            if pending[b] is not None:
                pending[b].wait()
            ob = outbufs[b]

            @plsc.parallel_loop(0, _OCHUNK, step=16, unroll=8)
            def _gather16(j):
                iv = idx_v[pl.ds(k * _OCHUNK + j, 16)]
                ob[pl.ds(j, 16)] = plsc.load_gather(col_v, [iv])

            pending[b] = pltpu.async_copy(
                ob, out_t_hbm.at[c, pl.ds(k * _OCHUNK, _OCHUNK)], osems[b]
            )
        if ci + 1 < _COLS_PER_W:
            col_cp = pltpu.async_copy(table_t_hbm.at[c + 1], col_v, sem_col)
            col_cp.wait()
    for p in pending:
        if p is not None:
            p.wait()


@jax.jit
def _sc_gather(table_t, idx):
    mesh = plsc.VectorSubcoreMesh(core_axis_name="c", subcore_axis_name="s")
    return pl.kernel(
        _gather_body,
        mesh=mesh,
        out_type=jax.ShapeDtypeStruct((D_MODEL, BATCH), jnp.float32),
        scratch_types=[
            pltpu.VMEM((MAX_LEN,), jnp.float32),
            pltpu.VMEM((BATCH,), jnp.int32),
            pltpu.VMEM((_OCHUNK,), jnp.float32),
            pltpu.VMEM((_OCHUNK,), jnp.float32),
            pltpu.SemaphoreType.DMA,
            pltpu.SemaphoreType.DMA,
            pltpu.SemaphoreType.DMA,
            pltpu.SemaphoreType.DMA,
        ],
        compiler_params=pltpu.CompilerParams(
            use_tc_tiling_on_sc=True, needs_layout_passes=False
        ),
    )(table_t, idx)


def kernel(window_position, window_size, pos_embedding):
    del window_size
    out_t = _sc_gather(pos_embedding.T, window_position.astype(jnp.int32))
    return out_t.T
